# Initial kernel scaffold; baseline (speedup 1.0000x reference)
import jax
import jax.numpy as jnp
from jax.experimental import pallas as pl


def kernel(g, h, e, params):
    raise NotImplementedError("write your pallas kernel here")


# speedup vs baseline: 15.9538x; 15.9538x over previous
import functools

import jax
import jax.numpy as jnp
import numpy as np
from jax import lax
from jax.experimental import pallas as pl
from jax.experimental.pallas import tpu as pltpu
from jax.experimental.pallas import tpu_sc as plsc

H = 256
HEADS = 8
DK = 32
N_REAL = 10000
NP = 10240
NE = 160000
UW = 384
ZW = 512
AC = 128
OUTP = 128

BN = 512
BE = 1000
GB = 200
SB = 200

_f32 = jnp.float32


def _full(shape):
    return pl.BlockSpec(shape, lambda i: (0,) * len(shape))


def _rows(width, blk):
    return pl.BlockSpec((blk, width), lambda i: (i, 0))


def _ln(x, g, b):
    mu = jnp.mean(x, axis=-1, keepdims=True)
    var = jnp.mean((x - mu) ** 2, axis=-1, keepdims=True)
    return (x - mu) * lax.rsqrt(var + 1e-5) * g + b


def _qkv_body(h_ref, wq, bq, wk, bk, wv, bv, q_o, k_o, v_o):
    hb = h_ref[...]
    q_o[...] = jnp.dot(hb, wq[...], preferred_element_type=_f32) + bq[...]
    k_o[...] = jnp.dot(hb, wk[...], preferred_element_type=_f32) + bk[...]
    v_o[...] = jnp.dot(hb, wv[...], preferred_element_type=_f32) + bv[...]


def _tc_qkv(h, wq, bq, wk, bk, wv320, bv320):
    return pl.pallas_call(
        _qkv_body,
        grid=(NP // BN,),
        in_specs=[_rows(H, BN), _full((H, H)), _full((1, H)),
                  _full((H, H)), _full((1, H)),
                  _full((H, UW)), _full((1, UW))],
        out_specs=[_rows(H, BN), _rows(H, BN), _rows(UW, BN)],
        out_shape=[jax.ShapeDtypeStruct((NP, H), _f32),
                   jax.ShapeDtypeStruct((NP, H), _f32),
                   jax.ShapeDtypeStruct((NP, UW), _f32)],
    )(h, wq, bq, wk, bk, wv320, bv320)


def _matbias_body(x_ref, w, b, o_ref):
    o_ref[...] = jnp.dot(x_ref[...], w[...], preferred_element_type=_f32) + b[...]


def _tc_matbias(x, w, b, blk):
    rows = x.shape[0]
    cols = w.shape[1]
    return pl.pallas_call(
        _matbias_body,
        grid=(rows // blk,),
        in_specs=[_rows(x.shape[1], blk), _full((x.shape[1], cols)),
                  _full((1, cols))],
        out_specs=_rows(cols, blk),
        out_shape=jax.ShapeDtypeStruct((rows, cols), _f32),
    )(x, w, b)


def _edge_body(ks_ref, qd_ref, ep_ref, vs_ref, e_ref,
               ow, ob, f1w, f1b, f2w, f2b, g1, b1, g2, b2,
               u_o, e2_o):
    sarr = ks_ref[...] * qd_ref[...] * ep_ref[...]
    ci = lax.broadcasted_iota(jnp.int32, (H, HEADS), 0) // DK
    hi = lax.broadcasted_iota(jnp.int32, (H, HEADS), 1)
    msum = (ci == hi).astype(_f32)
    att = jnp.exp(jnp.clip(
        jnp.dot(sarr, msum, preferred_element_type=_f32), -5.0, 5.0))
    hb2 = lax.broadcasted_iota(jnp.int32, (HEADS, UW), 0)
    cb2 = lax.broadcasted_iota(jnp.int32, (HEADS, UW), 1)
    mbc = (jnp.where(cb2 < H, cb2 // DK, cb2 - H) == hb2).astype(_f32)
    u_o[...] = jnp.dot(att, mbc, preferred_element_type=_f32) * vs_ref[...]
    e_o = jnp.dot(sarr, ow[...], preferred_element_type=_f32) + ob[...]
    e1 = _ln(e_ref[...] + e_o, g1[...], b1[...])
    ef = jnp.dot(
        jnp.maximum(jnp.dot(e1, f1w[...], preferred_element_type=_f32)
                    + f1b[...], 0.0),
        f2w[...], preferred_element_type=_f32) + f2b[...]
    e2_o[...] = _ln(e1 + ef, g2[...], b2[...])


def _tc_edge(ks, qd, ep, vs, e, ow, ob, f1w, f1b, f2w, f2b, g1, b1, g2, b2):
    return pl.pallas_call(
        _edge_body,
        grid=(NE // BE,),
        in_specs=[_rows(H, BE), _rows(H, BE), _rows(H, BE), _rows(UW, BE),
                  _rows(H, BE),
                  _full((H, H)), _full((1, H)),
                  _full((H, 2 * H)), _full((1, 2 * H)),
                  _full((2 * H, H)), _full((1, H)),
                  _full((1, H)), _full((1, H)), _full((1, H)), _full((1, H))],
        out_specs=[_rows(UW, BE), _rows(H, BE)],
        out_shape=[jax.ShapeDtypeStruct((NE, UW), _f32),
                   jax.ShapeDtypeStruct((NE, H), _f32)],
    )(ks, qd, ep, vs, e, ow, ob, f1w, f1b, f2w, f2b, g1, b1, g2, b2)


def _node_body(s_ref, h_ref, ow, ob, f1w, f1b, f2w, f2b, g1, b1, g2, b2, h2_o):
    sblk = s_ref[...]
    wv = sblk[:, :H]
    ri = lax.broadcasted_iota(jnp.int32, (ZW, H), 0)
    ci = lax.broadcasted_iota(jnp.int32, (ZW, H), 1)
    rh = jnp.where(ri >= H + AC, ri - H - AC, ri - H)
    mz = ((ri >= H) & (rh < HEADS) & (ci // DK == rh)).astype(_f32)
    zb = jnp.dot(sblk, mz, preferred_element_type=_f32) + 1e-6
    hat = wv / zb
    h_o = jnp.dot(hat, ow[...], preferred_element_type=_f32) + ob[...]
    h1 = _ln(h_ref[...] + h_o, g1[...], b1[...])
    hf = jnp.dot(
        jnp.maximum(jnp.dot(h1, f1w[...], preferred_element_type=_f32)
                    + f1b[...], 0.0),
        f2w[...], preferred_element_type=_f32) + f2b[...]
    h2_o[...] = _ln(h1 + hf, g2[...], b2[...])


def _tc_node(sacc, h, ow, ob, f1w, f1b, f2w, f2b, g1, b1, g2, b2):
    return pl.pallas_call(
        _node_body,
        grid=(NP // BN,),
        in_specs=[_rows(ZW, BN), _rows(H, BN),
                  _full((H, H)), _full((1, H)),
                  _full((H, 2 * H)), _full((1, 2 * H)),
                  _full((2 * H, H)), _full((1, H)),
                  _full((1, H)), _full((1, H)), _full((1, H)), _full((1, H))],
        out_specs=_rows(H, BN),
        out_shape=jax.ShapeDtypeStruct((NP, H), _f32),
    )(sacc, h, ow, ob, f1w, f1b, f2w, f2b, g1, b1, g2, b2)


def _sc_gather3(ktab, qtab, vtab, src, dst):
    mesh = plsc.VectorSubcoreMesh(core_axis_name="c", subcore_axis_name="s")
    ch = NE // 32
    nb = ch // GB

    @functools.partial(
        pl.kernel,
        out_type=[jax.ShapeDtypeStruct((NE, H), _f32),
                  jax.ShapeDtypeStruct((NE, H), _f32),
                  jax.ShapeDtypeStruct((NE, UW), _f32)],
        mesh=mesh,
        scratch_types=[
            pltpu.VMEM((GB,), jnp.int32),
            pltpu.VMEM((GB,), jnp.int32),
            pltpu.VMEM((GB, H), _f32),
            pltpu.VMEM((GB, UW), _f32),
            pltpu.SemaphoreType.DMA,
        ],
    )
    def kfn(k_hbm, q_hbm, v_hbm, src_hbm, dst_hbm, ok_hbm, oq_hbm, ov_hbm,
            srcv, dstv, rows, rows320, sem):
        wid = lax.axis_index("s") * 2 + lax.axis_index("c")
        base0 = wid * ch

        def body(j, carry):
            base = base0 + j * GB
            pltpu.sync_copy(src_hbm.at[pl.ds(base, GB)], srcv)
            pltpu.sync_copy(dst_hbm.at[pl.ds(base, GB)], dstv)
            pltpu.async_copy(k_hbm.at[srcv], rows, sem).wait()
            pltpu.sync_copy(rows, ok_hbm.at[pl.ds(base, GB)])
            pltpu.async_copy(q_hbm.at[dstv], rows, sem).wait()
            pltpu.sync_copy(rows, oq_hbm.at[pl.ds(base, GB)])
            pltpu.async_copy(v_hbm.at[srcv], rows320, sem).wait()
            pltpu.sync_copy(rows320, ov_hbm.at[pl.ds(base, GB)])
            return carry

        lax.fori_loop(0, nb, body, 0)

    return kfn(ktab, qtab, vtab, src, dst)


def _sc_scatter(u, dst, zinit):
    mesh = plsc.VectorSubcoreMesh(core_axis_name="c", subcore_axis_name="s")
    ch1 = NE // 16
    nb1 = ch1 // SB
    ch2 = NE // 32
    nb2 = ch2 // SB
    zr = NP // 16

    @functools.partial(
        pl.kernel,
        out_type=jax.ShapeDtypeStruct((NP, ZW), _f32),
        mesh=mesh,
        scratch_types=[
            pltpu.VMEM((SB,), jnp.int32),
            pltpu.VMEM((SB, AC), _f32),
            pltpu.VMEM_SHARED((NP, AC), _f32),
            pltpu.SemaphoreType.DMA,
        ],
    )
    def kfn(u_hbm, dst_hbm, z_hbm, out_hbm, idxv, stage, acc, sem):
        c = lax.axis_index("c")
        s = lax.axis_index("s")
        pltpu.sync_copy(z_hbm, acc.at[pl.ds(s * zr, zr)])
        plsc.subcore_barrier()

        def sweep(base0, nb, col0):
            def body(j, carry):
                base = base0 + j * SB
                pltpu.sync_copy(dst_hbm.at[pl.ds(base, SB)], idxv)
                pltpu.sync_copy(u_hbm.at[pl.ds(base, SB), pl.ds(col0, AC)],
                                stage)
                pltpu.sync_copy(stage, acc.at[idxv], add=True)
                return carry

            lax.fori_loop(0, nb, body, 0)

        def copyout(col0):
            pltpu.sync_copy(acc.at[pl.ds(s * zr, zr)],
                            out_hbm.at[pl.ds(s * zr, zr), pl.ds(col0, AC)])

        @pl.when(c == 0)
        def _():
            sweep(s * ch1, nb1, 0)

        @pl.when(c == 1)
        def _():
            sweep(s * ch1, nb1, AC)

        plsc.subcore_barrier()

        @pl.when(c == 0)
        def _():
            copyout(0)

        @pl.when(c == 1)
        def _():
            copyout(AC)

        pltpu.sync_copy(z_hbm, acc.at[pl.ds(s * zr, zr)])
        plsc.subcore_barrier()

        @pl.when(c == 0)
        def _():
            sweep(s * ch2, nb2, 2 * AC)

        @pl.when(c == 1)
        def _():
            sweep(NE // 2 + s * ch2, nb2, 2 * AC)

        plsc.subcore_barrier()

        @pl.when(c == 0)
        def _():
            copyout(2 * AC)

        @pl.when(c == 1)
        def _():
            copyout(3 * AC)

    return kfn(u, dst, zinit)


def kernel(g, h, e, params):
    src = g[0].astype(jnp.int32)
    dst = g[1].astype(jnp.int32)
    h = jnp.pad(h, ((0, NP - h.shape[0]), (0, 0)))
    zinit = jnp.zeros((NP // 16, AC), _f32)
    scale = np.float32(1.0 / np.sqrt(DK))

    def r(b):
        return b.reshape(1, -1)

    for p in params["layers"]:
        wv320 = jnp.pad(p["V"]["W"], ((0, 0), (0, UW - H)))
        bv320 = jnp.concatenate(
            [p["V"]["b"], jnp.ones((HEADS,), _f32),
             jnp.zeros((UW - H - HEADS,), _f32)]).reshape(1, UW)
        q, k, v = _tc_qkv(h, p["Q"]["W"], r(p["Q"]["b"]),
                          p["K"]["W"], r(p["K"]["b"]), wv320, bv320)
        ep = _tc_matbias(e, p["E"]["W"] * scale, r(p["E"]["b"]) * scale, BE)
        ks, qd, vs = _sc_gather3(k, q, v, src, dst)
        u, e = _tc_edge(ks, qd, ep, vs, e,
                        p["Oe"]["W"], r(p["Oe"]["b"]),
                        p["Fe1"]["W"], r(p["Fe1"]["b"]),
                        p["Fe2"]["W"], r(p["Fe2"]["b"]),
                        r(p["ln1e_g"]), r(p["ln1e_b"]),
                        r(p["ln2e_g"]), r(p["ln2e_b"]))
        sacc = _sc_scatter(u, dst, zinit)
        h = _tc_node(sacc, h,
                     p["Oh"]["W"], r(p["Oh"]["b"]),
                     p["Fh1"]["W"], r(p["Fh1"]["b"]),
                     p["Fh2"]["W"], r(p["Fh2"]["b"]),
                     r(p["ln1h_g"]), r(p["ln1h_b"]),
                     r(p["ln2h_g"]), r(p["ln2h_b"]))

    cw = jnp.pad(params["cls"]["W"], ((0, 0), (0, OUTP - 40)))
    cb = jnp.pad(params["cls"]["b"], ((0, OUTP - 40),)).reshape(1, OUTP)
    logits = _tc_matbias(h, cw, cb, BN)
    return logits[:N_REAL, :40]
